# Initial kernel scaffold; baseline (speedup 1.0000x reference)
import jax
import jax.numpy as jnp
from jax.experimental import pallas as pl


def kernel(catlist_0, catlist_1, cat_0, cat_1, cat_2, cat_3, cat_4, cat_5, cat_6, cat_7, cat_8, cat_9, cat_10, cat_11, cat_12, cont_0, cont_1, cont_2, cont_3, cont_4, cont_5, cont_6, cont_7, contembd_0, contembd_1, contembd_2, contembd_3, table_catlist_0, table_catlist_1, table_cat_0, table_cat_1, table_cat_2, table_cat_3, table_cat_4, table_cat_5, table_cat_6, table_cat_7, table_cat_8, table_cat_9, table_cat_10, table_cat_11, table_cat_12, W1, b1, W2, b2):
    raise NotImplementedError("write your pallas kernel here")


# speedup vs baseline: 24.0427x; 24.0427x over previous
import functools
import math

import jax
import jax.numpy as jnp
from jax import lax
from jax.experimental import pallas as pl
from jax.experimental.pallas import tpu as pltpu
from jax.experimental.pallas import tpu_sc as plsc

_B = 4096
_VOCAB = 100000
_NW = 32
_RPW = _B // _NW
_NFIELD = 113
_NDENSE = 72
_BN_SCALE = (1.0 + 1e-3) ** -0.5
_LANE = 16
_CHUNKS = _RPW // _LANE


def _body(table_hbm, idx_hbm, c_hbm, w_hbm, out_hbm,
          idx_v, vals_v, c_v, w_v, out_v, sem):
    wid = lax.axis_index("s") * 2 + lax.axis_index("c")
    base = wid * _RPW

    pltpu.sync_copy(idx_hbm.at[wid], idx_v)
    pltpu.sync_copy(c_hbm.at[wid], c_v)
    pltpu.sync_copy(w_hbm, w_v)

    def fire_body(j, carry):
        pltpu.async_copy(table_hbm.at[idx_v.at[j]], vals_v.at[j], sem)
        return carry

    lax.fori_loop(0, _NFIELD, fire_body, 0)

    zeros = tuple(jnp.zeros((_LANE,), jnp.float32) for _ in range(_CHUNKS))

    def dense_body(j, accs):
        w_b = w_v[j, pl.ds(0, _LANE)]
        return tuple(accs[k] + c_v[j, pl.ds(k * _LANE, _LANE)] * w_b
                     for k in range(_CHUNKS))

    dense_accs = lax.fori_loop(0, _NDENSE, dense_body, zeros)

    def drain_body(j, carry):
        pltpu.make_async_copy(table_hbm.at[idx_v.at[0]], vals_v.at[0],
                              sem).wait()
        return carry

    lax.fori_loop(0, _NFIELD, drain_body, 0)

    def red_body(j, accs):
        return tuple(accs[k] + vals_v[j, pl.ds(k * _LANE, _LANE)]
                     for k in range(_CHUNKS))

    cat_accs = lax.fori_loop(0, _NFIELD, red_body, zeros)

    bias = (w_v[_NDENSE, pl.ds(0, _LANE)]
            + w_v[_NDENSE + 1, pl.ds(0, _LANE)])
    for k in range(_CHUNKS):
        out_v[pl.ds(k * _LANE, _LANE)] = (
            cat_accs[k] + _BN_SCALE * dense_accs[k] + bias)

    pltpu.sync_copy(out_v, out_hbm.at[pl.ds(base, _RPW)])


@jax.jit
def _run(table, idx, c_all, w_all):
    mesh = plsc.VectorSubcoreMesh(core_axis_name="c", subcore_axis_name="s")
    kfn = pl.kernel(
        _body,
        mesh=mesh,
        out_type=jax.ShapeDtypeStruct((_B,), jnp.float32),
        scratch_types=[
            pltpu.VMEM((_NFIELD, _RPW), jnp.int32),
            pltpu.VMEM((_NFIELD, _RPW), jnp.float32),
            pltpu.VMEM((_NDENSE, _RPW), jnp.float32),
            pltpu.VMEM((80, _LANE), jnp.float32),
            pltpu.VMEM((_RPW,), jnp.float32),
            pltpu.SemaphoreType.DMA,
        ],
    )
    return kfn(table, idx, c_all, w_all)


def kernel(catlist_0, catlist_1, cat_0, cat_1, cat_2, cat_3, cat_4, cat_5,
           cat_6, cat_7, cat_8, cat_9, cat_10, cat_11, cat_12,
           cont_0, cont_1, cont_2, cont_3, cont_4, cont_5, cont_6, cont_7,
           contembd_0, contembd_1, contembd_2, contembd_3,
           table_catlist_0, table_catlist_1,
           table_cat_0, table_cat_1, table_cat_2, table_cat_3, table_cat_4,
           table_cat_5, table_cat_6, table_cat_7, table_cat_8, table_cat_9,
           table_cat_10, table_cat_11, table_cat_12,
           W1, b1, W2, b2):
    tables = [table_catlist_0, table_catlist_1,
              table_cat_0, table_cat_1, table_cat_2, table_cat_3, table_cat_4,
              table_cat_5, table_cat_6, table_cat_7, table_cat_8, table_cat_9,
              table_cat_10, table_cat_11, table_cat_12]
    table = jnp.concatenate([t.reshape(-1) for t in tables], axis=0)

    idx_parts = [catlist_0, catlist_1,
                 cat_0, cat_1, cat_2, cat_3, cat_4, cat_5, cat_6, cat_7,
                 cat_8, cat_9, cat_10, cat_11, cat_12]
    offs = jnp.concatenate([
        jnp.full((50,), 0, jnp.int32),
        jnp.full((50,), _VOCAB, jnp.int32),
        (jnp.arange(13, dtype=jnp.int32) + 2) * _VOCAB,
    ])
    idx = jnp.concatenate(idx_parts, axis=1) + offs[None, :]
    idx = idx.T.reshape(_NFIELD, _NW, _RPW).transpose(1, 0, 2)

    c_all = jnp.concatenate(
        [cont_0, cont_1, cont_2, cont_3, cont_4, cont_5, cont_6, cont_7,
         contembd_0, contembd_1, contembd_2, contembd_3], axis=1)
    c_all = c_all.T.reshape(_NDENSE, _NW, _RPW).transpose(1, 0, 2)

    w_all = jnp.concatenate(
        [W1.reshape(-1), W2.reshape(-1), b1.reshape(-1), b2.reshape(-1),
         jnp.zeros((6,), jnp.float32)])
    w_all = jnp.tile(w_all[:, None], (1, _LANE))

    out = _run(table, idx, c_all, w_all)
    return out.reshape(_B, 1)


# speedup vs baseline: 41.6212x; 1.7311x over previous
import functools
import math

import jax
import jax.numpy as jnp
from jax import lax
from jax.experimental import pallas as pl
from jax.experimental.pallas import tpu as pltpu
from jax.experimental.pallas import tpu_sc as plsc

_B = 4096
_NW = 32
_RPW = _B // _NW
_NLIST = 50
_NCAT = 13
_NFIELD = 2 * _NLIST + _NCAT
_NDENSE = 72
_BN_SCALE = (1.0 + 1e-3) ** -0.5
_LANE = 16
_CHUNKS = _RPW // _LANE


def _body(t_l0, t_l1, t_c0, t_c1, t_c2, t_c3, t_c4, t_c5, t_c6, t_c7, t_c8,
          t_c9, t_c10, t_c11, t_c12,
          idxl0_hbm, idxl1_hbm, idxc_hbm, c_hbm, w_hbm, out_hbm,
          idxl0_v, idxl1_v, idxc_v, vals_v, c_v, w_v, out_v, sem):
    wid = lax.axis_index("s") * 2 + lax.axis_index("c")
    base = wid * _RPW

    pltpu.sync_copy(idxl0_hbm.at[wid], idxl0_v)
    pltpu.sync_copy(idxl1_hbm.at[wid], idxl1_v)
    pltpu.sync_copy(idxc_hbm.at[wid], idxc_v)
    pltpu.sync_copy(c_hbm.at[wid], c_v)
    pltpu.sync_copy(w_hbm, w_v)

    def fire_l0(j, carry):
        pltpu.async_copy(t_l0.at[idxl0_v.at[j]], vals_v.at[j], sem)
        return carry

    lax.fori_loop(0, _NLIST, fire_l0, 0)

    def fire_l1(j, carry):
        pltpu.async_copy(t_l1.at[idxl1_v.at[j]], vals_v.at[_NLIST + j], sem)
        return carry

    lax.fori_loop(0, _NLIST, fire_l1, 0)

    for i, t in enumerate((t_c0, t_c1, t_c2, t_c3, t_c4, t_c5, t_c6, t_c7,
                           t_c8, t_c9, t_c10, t_c11, t_c12)):
        pltpu.async_copy(t.at[idxc_v.at[i]], vals_v.at[2 * _NLIST + i], sem)

    zeros = tuple(jnp.zeros((_LANE,), jnp.float32) for _ in range(_CHUNKS))

    def dense_body(j, accs):
        w_b = w_v[j, pl.ds(0, _LANE)]
        return tuple(accs[k] + c_v[j, pl.ds(k * _LANE, _LANE)] * w_b
                     for k in range(_CHUNKS))

    dense_accs = lax.fori_loop(0, _NDENSE, dense_body, zeros)

    def drain_body(j, carry):
        pltpu.make_async_copy(t_l0.at[idxl0_v.at[0]], vals_v.at[0],
                              sem).wait()
        return carry

    lax.fori_loop(0, _NFIELD, drain_body, 0)

    def red_body(j, accs):
        return tuple(accs[k] + vals_v[j, pl.ds(k * _LANE, _LANE)]
                     for k in range(_CHUNKS))

    cat_accs = lax.fori_loop(0, _NFIELD, red_body, zeros)

    bias = (w_v[_NDENSE, pl.ds(0, _LANE)]
            + w_v[_NDENSE + 1, pl.ds(0, _LANE)])
    for k in range(_CHUNKS):
        out_v[pl.ds(k * _LANE, _LANE)] = (
            cat_accs[k] + _BN_SCALE * dense_accs[k] + bias)

    pltpu.sync_copy(out_v, out_hbm.at[pl.ds(base, _RPW)])


@jax.jit
def _run(tables, idxl0, idxl1, idxc, c_all, w_all):
    mesh = plsc.VectorSubcoreMesh(core_axis_name="c", subcore_axis_name="s")
    kfn = pl.kernel(
        _body,
        mesh=mesh,
        out_type=jax.ShapeDtypeStruct((_B,), jnp.float32),
        scratch_types=[
            pltpu.VMEM((_NLIST, _RPW), jnp.int32),
            pltpu.VMEM((_NLIST, _RPW), jnp.int32),
            pltpu.VMEM((_NCAT, _RPW), jnp.int32),
            pltpu.VMEM((_NFIELD, _RPW), jnp.float32),
            pltpu.VMEM((_NDENSE, _RPW), jnp.float32),
            pltpu.VMEM((80, _LANE), jnp.float32),
            pltpu.VMEM((_RPW,), jnp.float32),
            pltpu.SemaphoreType.DMA,
        ],
    )
    return kfn(*tables, idxl0, idxl1, idxc, c_all, w_all)


def kernel(catlist_0, catlist_1, cat_0, cat_1, cat_2, cat_3, cat_4, cat_5,
           cat_6, cat_7, cat_8, cat_9, cat_10, cat_11, cat_12,
           cont_0, cont_1, cont_2, cont_3, cont_4, cont_5, cont_6, cont_7,
           contembd_0, contembd_1, contembd_2, contembd_3,
           table_catlist_0, table_catlist_1,
           table_cat_0, table_cat_1, table_cat_2, table_cat_3, table_cat_4,
           table_cat_5, table_cat_6, table_cat_7, table_cat_8, table_cat_9,
           table_cat_10, table_cat_11, table_cat_12,
           W1, b1, W2, b2):
    tables = [t.reshape(-1) for t in
              (table_catlist_0, table_catlist_1,
               table_cat_0, table_cat_1, table_cat_2, table_cat_3,
               table_cat_4, table_cat_5, table_cat_6, table_cat_7,
               table_cat_8, table_cat_9, table_cat_10, table_cat_11,
               table_cat_12)]

    idxl0 = catlist_0.T.reshape(_NLIST, _NW, _RPW).transpose(1, 0, 2)
    idxl1 = catlist_1.T.reshape(_NLIST, _NW, _RPW).transpose(1, 0, 2)
    idxc = jnp.concatenate(
        [cat_0, cat_1, cat_2, cat_3, cat_4, cat_5, cat_6, cat_7, cat_8,
         cat_9, cat_10, cat_11, cat_12], axis=1)
    idxc = idxc.T.reshape(_NCAT, _NW, _RPW).transpose(1, 0, 2)

    c_all = jnp.concatenate(
        [cont_0, cont_1, cont_2, cont_3, cont_4, cont_5, cont_6, cont_7,
         contembd_0, contembd_1, contembd_2, contembd_3], axis=1)
    c_all = c_all.T.reshape(_NDENSE, _NW, _RPW).transpose(1, 0, 2)

    w_all = jnp.concatenate(
        [W1.reshape(-1), W2.reshape(-1), b1.reshape(-1), b2.reshape(-1),
         jnp.zeros((6,), jnp.float32)])
    w_all = jnp.tile(w_all[:, None], (1, _LANE))

    out = _run(tables, idxl0, idxl1, idxc, c_all, w_all)
    return out.reshape(_B, 1)


# speedup vs baseline: 42.4939x; 1.0210x over previous
import functools
import math

import jax
import jax.numpy as jnp
from jax import lax
from jax.experimental import pallas as pl
from jax.experimental.pallas import tpu as pltpu
from jax.experimental.pallas import tpu_sc as plsc

_B = 4096
_NW = 32
_RPW = _B // _NW
_NLIST = 50
_NCAT = 13
_NFIELD = 2 * _NLIST + _NCAT
_NCONT = 8
_NEMB = 64
_NDENSE = _NCONT + _NEMB
_BN_SCALE = (1.0 + 1e-3) ** -0.5
_LANE = 16
_CHUNKS = _RPW // _LANE


def _body(t_l0, t_l1, t_c0, t_c1, t_c2, t_c3, t_c4, t_c5, t_c6, t_c7, t_c8,
          t_c9, t_c10, t_c11, t_c12,
          il0_hbm, il1_hbm,
          s0_hbm, s1_hbm, s2_hbm, s3_hbm, s4_hbm, s5_hbm, s6_hbm, s7_hbm,
          s8_hbm, s9_hbm, s10_hbm, s11_hbm, s12_hbm,
          n0_hbm, n1_hbm, n2_hbm, n3_hbm, n4_hbm, n5_hbm, n6_hbm, n7_hbm,
          ce_hbm, w_hbm, out_hbm,
          idxl0_v, idxl1_v, idxc_v, vals_v, cont_v, ce_v, w_v, out_v,
          sem_l0, sem_l1, sem_idxc, sem_dense, sem_g):
    wid = lax.axis_index("s") * 2 + lax.axis_index("c")
    base = wid * _RPW
    rows = pl.ds(base, _RPW)

    cp_l0 = pltpu.async_copy(il0_hbm.at[wid], idxl0_v, sem_l0)
    cp_l1 = pltpu.async_copy(il1_hbm.at[wid], idxl1_v, sem_l1)
    singles = (s0_hbm, s1_hbm, s2_hbm, s3_hbm, s4_hbm, s5_hbm, s6_hbm,
               s7_hbm, s8_hbm, s9_hbm, s10_hbm, s11_hbm, s12_hbm)
    cp_idxc = [pltpu.async_copy(s.at[rows], idxc_v.at[i], sem_idxc)
               for i, s in enumerate(singles)]
    conts = (n0_hbm, n1_hbm, n2_hbm, n3_hbm, n4_hbm, n5_hbm, n6_hbm, n7_hbm)
    cp_dense = [pltpu.async_copy(c.at[rows], cont_v.at[i], sem_dense)
                for i, c in enumerate(conts)]
    cp_dense.append(pltpu.async_copy(ce_hbm.at[wid], ce_v, sem_dense))
    cp_dense.append(pltpu.async_copy(w_hbm, w_v, sem_dense))

    cp_l0.wait()

    def fire_l0(j, carry):
        pltpu.async_copy(t_l0.at[idxl0_v.at[j]], vals_v.at[j], sem_g)
        return carry

    lax.fori_loop(0, _NLIST, fire_l0, 0)
    cp_l1.wait()

    def fire_l1(j, carry):
        pltpu.async_copy(t_l1.at[idxl1_v.at[j]], vals_v.at[_NLIST + j], sem_g)
        return carry

    lax.fori_loop(0, _NLIST, fire_l1, 0)

    for cp in cp_idxc:
        cp.wait()
    for i, t in enumerate((t_c0, t_c1, t_c2, t_c3, t_c4, t_c5, t_c6, t_c7,
                           t_c8, t_c9, t_c10, t_c11, t_c12)):
        pltpu.async_copy(t.at[idxc_v.at[i]], vals_v.at[2 * _NLIST + i], sem_g)

    for cp in cp_dense:
        cp.wait()

    zeros = tuple(jnp.zeros((_LANE,), jnp.float32) for _ in range(_CHUNKS))

    def cont_body(j, accs):
        w_b = w_v[j, pl.ds(0, _LANE)]
        return tuple(accs[k] + cont_v[j, pl.ds(k * _LANE, _LANE)] * w_b
                     for k in range(_CHUNKS))

    dense_accs = lax.fori_loop(0, _NCONT, cont_body, zeros)

    def emb_body(j, accs):
        w_b = w_v[_NCONT + j, pl.ds(0, _LANE)]
        return tuple(accs[k] + ce_v[j, pl.ds(k * _LANE, _LANE)] * w_b
                     for k in range(_CHUNKS))

    dense_accs = lax.fori_loop(0, _NEMB, emb_body, dense_accs)

    def drain_body(j, carry):
        pltpu.make_async_copy(t_l0.at[idxl0_v.at[0]], vals_v.at[0],
                              sem_g).wait()
        return carry

    lax.fori_loop(0, _NFIELD, drain_body, 0)

    def red_body(j, accs):
        return tuple(accs[k] + vals_v[j, pl.ds(k * _LANE, _LANE)]
                     for k in range(_CHUNKS))

    cat_accs = lax.fori_loop(0, _NFIELD, red_body, zeros)

    bias = (w_v[_NDENSE, pl.ds(0, _LANE)]
            + w_v[_NDENSE + 1, pl.ds(0, _LANE)])
    for k in range(_CHUNKS):
        out_v[pl.ds(k * _LANE, _LANE)] = (
            cat_accs[k] + _BN_SCALE * dense_accs[k] + bias)

    pltpu.sync_copy(out_v, out_hbm.at[pl.ds(base, _RPW)])


@jax.jit
def _run(tables, il0, il1, singles, conts, ce, w_all):
    mesh = plsc.VectorSubcoreMesh(core_axis_name="c", subcore_axis_name="s")
    kfn = pl.kernel(
        _body,
        mesh=mesh,
        out_type=jax.ShapeDtypeStruct((_B,), jnp.float32),
        scratch_types=[
            pltpu.VMEM((_NLIST, _RPW), jnp.int32),
            pltpu.VMEM((_NLIST, _RPW), jnp.int32),
            pltpu.VMEM((_NCAT, _RPW), jnp.int32),
            pltpu.VMEM((_NFIELD, _RPW), jnp.float32),
            pltpu.VMEM((_NCONT, _RPW), jnp.float32),
            pltpu.VMEM((_NEMB, _RPW), jnp.float32),
            pltpu.VMEM((80, _LANE), jnp.float32),
            pltpu.VMEM((_RPW,), jnp.float32),
            pltpu.SemaphoreType.DMA,
            pltpu.SemaphoreType.DMA,
            pltpu.SemaphoreType.DMA,
            pltpu.SemaphoreType.DMA,
            pltpu.SemaphoreType.DMA,
        ],
    )
    return kfn(*tables, il0, il1, *singles, *conts, ce, w_all)


def kernel(catlist_0, catlist_1, cat_0, cat_1, cat_2, cat_3, cat_4, cat_5,
           cat_6, cat_7, cat_8, cat_9, cat_10, cat_11, cat_12,
           cont_0, cont_1, cont_2, cont_3, cont_4, cont_5, cont_6, cont_7,
           contembd_0, contembd_1, contembd_2, contembd_3,
           table_catlist_0, table_catlist_1,
           table_cat_0, table_cat_1, table_cat_2, table_cat_3, table_cat_4,
           table_cat_5, table_cat_6, table_cat_7, table_cat_8, table_cat_9,
           table_cat_10, table_cat_11, table_cat_12,
           W1, b1, W2, b2):
    tables = [t.reshape(-1) for t in
              (table_catlist_0, table_catlist_1,
               table_cat_0, table_cat_1, table_cat_2, table_cat_3,
               table_cat_4, table_cat_5, table_cat_6, table_cat_7,
               table_cat_8, table_cat_9, table_cat_10, table_cat_11,
               table_cat_12)]
    singles = [c.reshape(-1) for c in
               (cat_0, cat_1, cat_2, cat_3, cat_4, cat_5, cat_6, cat_7,
                cat_8, cat_9, cat_10, cat_11, cat_12)]
    conts = [c.reshape(-1) for c in
             (cont_0, cont_1, cont_2, cont_3, cont_4, cont_5, cont_6,
              cont_7)]

    il0 = catlist_0.reshape(_NW, _RPW, _NLIST).transpose(0, 2, 1)
    il1 = catlist_1.reshape(_NW, _RPW, _NLIST).transpose(0, 2, 1)
    ce = jnp.concatenate(
        [contembd_0, contembd_1, contembd_2, contembd_3], axis=1)
    ce = ce.reshape(_NW, _RPW, _NEMB).transpose(0, 2, 1)

    w_all = jnp.concatenate(
        [W1.reshape(-1), W2.reshape(-1), b1.reshape(-1), b2.reshape(-1),
         jnp.zeros((6,), jnp.float32)])
    w_all = jnp.tile(w_all[:, None], (1, _LANE))

    out = _run(tables, il0, il1, singles, conts, ce, w_all)
    return out.reshape(_B, 1)
